# baseline (speedup 1.0000x reference)
import functools

import jax
import jax.numpy as jnp
from jax import lax
from jax.experimental import pallas as pl
from jax.experimental.pallas import tpu as pltpu
from jax.experimental.pallas import tpu_sc as plsc

GAMMA = 0.1
TAO = 10.0
B_CONST = 1.0
BETA = 1.0
LAMBDA_ = 0.1
EPS = 1e-06

K = 8192
D = 256
L = 16
NC = 2
NS = 16
NW = NC * NS
RPW = K // NW
NGRP = RPW // L

_mesh = plsc.VectorSubcoreMesh(core_axis_name="c", subcore_axis_name="s")


@functools.partial(
    pl.kernel,
    out_type=jax.ShapeDtypeStruct((K,), jnp.float32),
    mesh=_mesh,
    scratch_types=[
        pltpu.VMEM((D,), jnp.float32),
        pltpu.VMEM((RPW * D,), jnp.float32),
        pltpu.VMEM((RPW,), jnp.float32),
    ],
    compiler_params=pltpu.CompilerParams(needs_layout_passes=False),
)
def _sc_d2(c_hbm, p_hbm, out_hbm, c_v, buf, out_v):
    wid = lax.axis_index("s") * NC + lax.axis_index("c")
    base = wid * RPW
    pltpu.sync_copy(c_hbm, c_v)
    pltpu.sync_copy(p_hbm.at[pl.ds(base * D, RPW * D)], buf)

    def group_body(g, carry):
        idx0 = (g * L + lax.iota(jnp.int32, L)) * D
        acc = jnp.zeros((L,), jnp.float32)
        for jb in range(D // L):
            cvec = c_v[pl.ds(jb * L, L)]
            for jj in range(L):
                vals = plsc.load_gather(buf, [idx0 + (jb * L + jj)])
                dlt = cvec[jj] - vals
                acc = acc + dlt * dlt
        out_v[pl.ds(g * L, L)] = acc
        return carry

    lax.fori_loop(0, NGRP, group_body, 0)
    pltpu.sync_copy(out_v, out_hbm.at[pl.ds(base, RPW)])


def _tc_epilogue_body(label_ref, d2_ref, lab_ref, loss_ref, mind_ref):
    label = label_ref[0, 0]
    d2 = d2_ref[...]
    mask = lab_ref[...] == label
    d = jnp.sqrt(d2)
    e = jnp.exp(-GAMMA * d2)
    one = jnp.sum(e)
    num = jnp.sum(jnp.where(mask, e, 0.0))
    dce = -jnp.log(num / one)
    g1 = jnp.log1p(jnp.exp(B_CONST - (TAO - d)))
    g2 = jnp.log1p(jnp.exp(B_CONST + (TAO - d)))
    pw = jnp.sum(jnp.where(mask, g1, 0.0)) + jnp.sum(g2)
    mind2 = jnp.min(jnp.where(mask, d2, jnp.inf))
    loss_ref[0, 0] = dce + LAMBDA_ * pw
    mind_ref[0, 0] = jnp.sqrt(mind2)


_tc_epilogue = pl.pallas_call(
    _tc_epilogue_body,
    out_shape=(
        jax.ShapeDtypeStruct((1, 1), jnp.float32),
        jax.ShapeDtypeStruct((1, 1), jnp.float32),
    ),
    in_specs=[
        pl.BlockSpec(memory_space=pltpu.SMEM),
        pl.BlockSpec(memory_space=pltpu.VMEM),
        pl.BlockSpec(memory_space=pltpu.VMEM),
    ],
    out_specs=(
        pl.BlockSpec(memory_space=pltpu.SMEM),
        pl.BlockSpec(memory_space=pltpu.SMEM),
    ),
)


def kernel(feature, label, prototypes, proto_labels):
    c = feature.reshape(D).astype(jnp.float32) + jnp.float32(EPS)
    p_flat = prototypes.reshape(K * D)
    d2 = _sc_d2(c, p_flat)
    lab2d = proto_labels.astype(jnp.int32).reshape(64, 128)
    label2d = jnp.asarray(label, jnp.int32).reshape(1, 1)
    loss, mind = _tc_epilogue(label2d, d2.reshape(64, 128), lab2d)
    return (loss.reshape(()), mind.reshape(()))


# speedup vs baseline: 2.0681x; 2.0681x over previous
import functools

import jax
import jax.numpy as jnp
from jax import lax
from jax.experimental import pallas as pl
from jax.experimental.pallas import tpu as pltpu
from jax.experimental.pallas import tpu_sc as plsc

GAMMA = 0.1
TAO = 10.0
B_CONST = 1.0
BETA = 1.0
LAMBDA_ = 0.1
EPS = 1e-06

K = 8192
D = 256
L = 16
NC = 2
NS = 16
NW = NC * NS
RPW = K // NW
CHUNK = 64
NCHUNK = RPW // CHUNK

_mesh = plsc.VectorSubcoreMesh(core_axis_name="c", subcore_axis_name="s")

_LANE_IOTA = None


@functools.partial(
    pl.kernel,
    out_type=jax.ShapeDtypeStruct((K,), jnp.float32),
    mesh=_mesh,
    scratch_types=[
        pltpu.VMEM((D,), jnp.float32),
        pltpu.VMEM((CHUNK, D), jnp.float32),
        pltpu.VMEM((CHUNK, D), jnp.float32),
        pltpu.VMEM((RPW,), jnp.float32),
        pltpu.SemaphoreType.DMA,
        pltpu.SemaphoreType.DMA,
    ],
    compiler_params=pltpu.CompilerParams(needs_layout_passes=False),
)
def _sc_d2(c_hbm, p_hbm, out_hbm, c_v, buf0, buf1, out_v, sem0, sem1):
    wid = lax.axis_index("s") * NC + lax.axis_index("c")
    base = wid * RPW
    pltpu.sync_copy(c_hbm, c_v)

    bufs = (buf0, buf1)
    sems = (sem0, sem1)
    copies = [None] * NCHUNK
    copies[0] = pltpu.async_copy(
        p_hbm.at[pl.ds(base, CHUNK)], buf0, sem0)

    cvecs = [c_v[pl.ds(jb * L, L)] for jb in range(D // L)]
    lane = lax.iota(jnp.int32, L)

    for chunk in range(NCHUNK):
        buf = bufs[chunk % 2]
        if chunk + 1 < NCHUNK:
            copies[chunk + 1] = pltpu.async_copy(
                p_hbm.at[pl.ds(base + (chunk + 1) * CHUNK, CHUNK)],
                bufs[(chunk + 1) % 2], sems[(chunk + 1) % 2])
        copies[chunk].wait()

        def group_body(g, carry, buf=buf, chunk=chunk):
            rowsums = jnp.zeros((L,), jnp.float32)
            for rr in range(L):
                row = g * L + rr
                acc = jnp.zeros((L,), jnp.float32)
                for jb in range(D // L):
                    v = buf[row, pl.ds(jb * L, L)]
                    dlt = cvecs[jb] - v
                    acc = acc + dlt * dlt
                s = jnp.sum(acc)
                rowsums = jnp.where(lane == rr, s, rowsums)
            out_v[pl.ds(chunk * CHUNK + g * L, L)] = rowsums
            return carry

        lax.fori_loop(0, CHUNK // L, group_body, 0)

    pltpu.sync_copy(out_v, out_hbm.at[pl.ds(base, RPW)])


def _tc_epilogue_body(label_ref, d2_ref, lab_ref, loss_ref, mind_ref):
    label = label_ref[0, 0]
    d2 = d2_ref[...]
    mask = lab_ref[...] == label
    d = jnp.sqrt(d2)
    e = jnp.exp(-GAMMA * d2)
    one = jnp.sum(e)
    num = jnp.sum(jnp.where(mask, e, 0.0))
    dce = -jnp.log(num / one)
    g1 = jnp.log1p(jnp.exp(B_CONST - (TAO - d)))
    g2 = jnp.log1p(jnp.exp(B_CONST + (TAO - d)))
    pw = jnp.sum(jnp.where(mask, g1, 0.0)) + jnp.sum(g2)
    mind2 = jnp.min(jnp.where(mask, d2, jnp.inf))
    loss_ref[0, 0] = dce + LAMBDA_ * pw
    mind_ref[0, 0] = jnp.sqrt(mind2)


_tc_epilogue = pl.pallas_call(
    _tc_epilogue_body,
    out_shape=(
        jax.ShapeDtypeStruct((1, 1), jnp.float32),
        jax.ShapeDtypeStruct((1, 1), jnp.float32),
    ),
    in_specs=[
        pl.BlockSpec(memory_space=pltpu.SMEM),
        pl.BlockSpec(memory_space=pltpu.VMEM),
        pl.BlockSpec(memory_space=pltpu.VMEM),
    ],
    out_specs=(
        pl.BlockSpec(memory_space=pltpu.SMEM),
        pl.BlockSpec(memory_space=pltpu.SMEM),
    ),
)


def kernel(feature, label, prototypes, proto_labels):
    c = feature.reshape(D).astype(jnp.float32) + jnp.float32(EPS)
    d2 = _sc_d2(c, prototypes)
    lab2d = proto_labels.astype(jnp.int32).reshape(64, 128)
    label2d = jnp.asarray(label, jnp.int32).reshape(1, 1)
    loss, mind = _tc_epilogue(label2d, d2.reshape(64, 128), lab2d)
    return (loss.reshape(()), mind.reshape(()))


# speedup vs baseline: 5.9025x; 2.8541x over previous
import functools

import jax
import jax.numpy as jnp
from jax import lax
from jax.experimental import pallas as pl
from jax.experimental.pallas import tpu as pltpu
from jax.experimental.pallas import tpu_sc as plsc

GAMMA = 0.1
TAO = 10.0
B_CONST = 1.0
BETA = 1.0
LAMBDA_ = 0.1
EPS = 1e-06

K = 8192
D = 256
L = 16
NC = 2
NS = 16
NW = NC * NS
RPW = K // NW
CHUNK = 64
NCHUNK = RPW // CHUNK

BLK = 1024
NBLK = K // BLK

_mesh = plsc.VectorSubcoreMesh(core_axis_name="c", subcore_axis_name="s")


@functools.partial(
    pl.kernel,
    out_type=jax.ShapeDtypeStruct((K,), jnp.float32),
    mesh=_mesh,
    scratch_types=[
        pltpu.VMEM((D,), jnp.float32),
        pltpu.VMEM((CHUNK, D), jnp.float32),
        pltpu.VMEM((CHUNK, D), jnp.float32),
        pltpu.VMEM((RPW,), jnp.float32),
        pltpu.SemaphoreType.DMA,
        pltpu.SemaphoreType.DMA,
    ],
    compiler_params=pltpu.CompilerParams(needs_layout_passes=False),
)
def _sc_d2(c_hbm, p_hbm, out_hbm, c_v, buf0, buf1, out_v, sem0, sem1):
    wid = lax.axis_index("s") * NC + lax.axis_index("c")
    base = wid * RPW
    pltpu.sync_copy(c_hbm, c_v)

    bufs = (buf0, buf1)
    sems = (sem0, sem1)
    copies = [None] * NCHUNK
    copies[0] = pltpu.async_copy(
        p_hbm.at[pl.ds(base, CHUNK)], buf0, sem0)

    cvecs = [c_v[pl.ds(jb * L, L)] for jb in range(D // L)]
    lane = lax.iota(jnp.int32, L)

    for chunk in range(NCHUNK):
        buf = bufs[chunk % 2]
        if chunk + 1 < NCHUNK:
            copies[chunk + 1] = pltpu.async_copy(
                p_hbm.at[pl.ds(base + (chunk + 1) * CHUNK, CHUNK)],
                bufs[(chunk + 1) % 2], sems[(chunk + 1) % 2])
        copies[chunk].wait()

        def group_body(g, carry, buf=buf, chunk=chunk):
            rowsums = jnp.zeros((L,), jnp.float32)
            for rr in range(L):
                row = g * L + rr
                acc = jnp.zeros((L,), jnp.float32)
                for jb in range(D // L):
                    v = buf[row, pl.ds(jb * L, L)]
                    dlt = cvecs[jb] - v
                    acc = acc + dlt * dlt
                s = jnp.sum(acc)
                rowsums = jnp.where(lane == rr, s, rowsums)
            out_v[pl.ds(chunk * CHUNK + g * L, L)] = rowsums
            return carry

        lax.fori_loop(0, CHUNK // L, group_body, 0)

    pltpu.sync_copy(out_v, out_hbm.at[pl.ds(base, RPW)])


def _softplus(z):
    return jnp.log1p(jnp.exp(z))


def _tc_full_body(label_ref, c_ref, p_ref, lab_ref,
                  loss_ref, mind_ref, d2s):
    pid = pl.program_id(0)

    diff = c_ref[...] - p_ref[...]
    sq = diff * diff
    ones = jnp.ones((D, 1), jnp.float32)
    d2s[pl.ds(pid * (BLK // 128), BLK // 128), :] = jax.lax.dot_general(
        sq, ones, (((1,), (0,)), ((), ())),
        preferred_element_type=jnp.float32).reshape(BLK // 128, 128)

    @pl.when(pid == NBLK - 1)
    def _fin():
        d2 = d2s[...]
        mask = lab_ref[...] == label_ref[0, 0]
        d = jnp.sqrt(d2)
        e = jnp.exp(-GAMMA * d2)
        one = jnp.sum(e)
        num = jnp.sum(jnp.where(mask, e, 0.0))
        g1 = _softplus(B_CONST - (TAO - d))
        g2 = _softplus(B_CONST + (TAO - d))
        pw = jnp.sum(jnp.where(mask, g1, 0.0)) + jnp.sum(g2)
        mind2 = jnp.min(jnp.where(mask, d2, jnp.inf))
        dce = -jnp.log(num / one)
        loss_ref[0, 0] = dce + LAMBDA_ * pw
        mind_ref[0, 0] = jnp.sqrt(mind2)


_tc_full = pl.pallas_call(
    _tc_full_body,
    grid=(NBLK,),
    in_specs=[
        pl.BlockSpec(memory_space=pltpu.SMEM),
        pl.BlockSpec((1, D), lambda i: (0, 0)),
        pl.BlockSpec((BLK, D), lambda i: (i, 0)),
        pl.BlockSpec((K // 128, 128), lambda i: (0, 0)),
    ],
    out_specs=(
        pl.BlockSpec(memory_space=pltpu.SMEM),
        pl.BlockSpec(memory_space=pltpu.SMEM),
    ),
    out_shape=(
        jax.ShapeDtypeStruct((1, 1), jnp.float32),
        jax.ShapeDtypeStruct((1, 1), jnp.float32),
    ),
    scratch_shapes=[pltpu.VMEM((K // 128, 128), jnp.float32)],
)


def kernel(feature, label, prototypes, proto_labels):
    c = feature.astype(jnp.float32) + jnp.float32(EPS)
    lab = proto_labels.astype(jnp.int32).reshape(K // 128, 128)
    label2d = jnp.asarray(label, jnp.int32).reshape(1, 1)
    loss, mind = _tc_full(label2d, c, prototypes, lab)
    return (loss.reshape(()), mind.reshape(()))
